# Initial kernel scaffold; baseline (speedup 1.0000x reference)
import jax
import jax.numpy as jnp
from jax.experimental import pallas as pl


def kernel(x, current_level, target_level, parent_map, W_down, b_down):
    raise NotImplementedError("write your pallas kernel here")


# speedup vs baseline: 27.8271x; 27.8271x over previous
import jax
import jax.numpy as jnp
from jax import lax
from jax.experimental import pallas as pl
from jax.experimental.pallas import tpu as pltpu
from jax.experimental.pallas import tpu_sc as plsc

_NC = 2
_NS = 16
_NW = _NC * _NS

_P = 256
_CHUNK = 128
_NSUB = _CHUNK // 128


def _vext(ref, i):
    return ref[pl.ds(i * 16, 16)][0]


def _sc_segment_sums(xf, pm_pad, bounds, bt, n, nl, d):
    nblk = nl // _P
    items = bt * nblk
    per_w = items // _NW

    def body(xf_hbm, pm_hbm, bounds_hbm, sums_hbm, cnts_hbm,
             acc_sh, cnt_sh, chunk, zbuf, pmbuf, idx2, onesb, boundsv):
        cid = lax.axis_index("c")
        sid = lax.axis_index("s")
        wid = sid * _NC + cid
        pltpu.sync_copy(bounds_hbm, boundsv)

        ones16 = jnp.full((16,), 1.0, jnp.float32)
        zeros16 = jnp.zeros((16,), jnp.float32)

        @pl.loop(0, 128)
        def _fill_ones(r):
            for g in range(d // 16):
                onesb[r, pl.ds(g * 16, 16)] = ones16

        @pl.loop(0, (_P + 16) // 4)
        def _fill_zeros(r):
            for g in range(d // 16):
                zbuf[r, pl.ds(g * 16, 16)] = zeros16

        bank = _P + 16
        base = sid * bank

        for k in range(per_w):
            t = wid * per_w + k
            b = t // nblk
            pb = t - b * nblk
            p0 = pb * _P
            r0 = _vext(boundsv, pb)
            r1 = _vext(boundsv, pb + 1)
            do_cnt = ((b == 0) & (pb < nblk // 2)) | ((b == 1) & (pb >= nblk // 2))

            quarter = bank // 4
            for q in range(4):
                pltpu.sync_copy(zbuf, acc_sh.at[pl.ds(base + q * quarter, quarter)])

            @pl.when(do_cnt)
            def _zero_cnt():
                for q in range(4):
                    pltpu.sync_copy(zbuf, cnt_sh.at[pl.ds(base + q * quarter, quarter)])

            a0 = (r0 // 8) * 8
            nch = (r1 - a0 + _CHUNK - 1) // _CHUNK
            rowbase = b * n

            @pl.loop(0, nch)
            def _chunk(ch):
                st = a0 + ch * _CHUNK
                gst = jnp.minimum(rowbase + st, bt * n - _CHUNK)
                st_c = gst - rowbase
                pltpu.sync_copy(pm_hbm.at[pl.ds(st_c, _CHUNK)], pmbuf)
                pltpu.sync_copy(xf_hbm.at[pl.ds(gst, _CHUNK)], chunk)
                vlo = rowbase + jnp.maximum(st, r0)
                vhi = rowbase + r1
                for j16 in range(_CHUNK // 16):
                    pv = pmbuf[pl.ds(j16 * 16, 16)]
                    rows = gst + j16 * 16 + lax.iota(jnp.int32, 16)
                    valid = (rows >= vlo) & (rows < vhi)
                    li = base + jnp.where(valid, pv - p0, _P)
                    idx2[j16 // 8, pl.ds((j16 % 8) * 16, 16)] = li
                for j in range(_NSUB):
                    pltpu.sync_copy(chunk.at[pl.ds(j * 128, 128)],
                                    acc_sh.at[idx2.at[j]], add=True)

                    @pl.when(do_cnt)
                    def _scatter_cnt():
                        pltpu.sync_copy(onesb, cnt_sh.at[idx2.at[j]], add=True)

            q0 = b * nl + p0
            pltpu.sync_copy(acc_sh.at[pl.ds(base, _P)],
                            sums_hbm.at[pl.ds(q0, _P)])

            @pl.when(do_cnt)
            def _store_counts():
                pltpu.sync_copy(cnt_sh.at[pl.ds(base, _P)],
                                cnts_hbm.at[pl.ds(p0, _P)])

    mesh = plsc.VectorSubcoreMesh(core_axis_name="c", subcore_axis_name="s")
    return pl.kernel(
        body,
        out_type=(
            jax.ShapeDtypeStruct((bt * nl, d), jnp.float32),
            jax.ShapeDtypeStruct((nl, d), jnp.float32),
        ),
        mesh=mesh,
        scratch_types=[
            pltpu.VMEM_SHARED((_NS * (_P + 16), d), jnp.float32),
            pltpu.VMEM_SHARED((_NS * (_P + 16), d), jnp.float32),
            pltpu.VMEM((_CHUNK, d), jnp.float32),
            pltpu.VMEM(((_P + 16) // 4, d), jnp.float32),
            pltpu.VMEM((_CHUNK,), jnp.int32),
            pltpu.VMEM((_NSUB, 128), jnp.int32),
            pltpu.VMEM((128, d), jnp.float32),
            pltpu.VMEM(((nl // _P + 1) * 16,), jnp.int32),
        ],
    )(xf, pm_pad, bounds)


def _tc_mean_conv(sums, cnts, w, bias, bt, nl, d):
    nblk = nl // _P

    def body(s_ref, c_ref, w_ref, b_ref, o_ref):
        s = s_ref[0]
        cnt = jnp.maximum(c_ref[:, 0:1], 1e-6)
        r = s / cnt
        y = lax.dot_general(r, w_ref[...], (((1,), (1,)), ((), ())),
                            preferred_element_type=jnp.float32)
        o_ref[0] = y + b_ref[...]

    return pl.pallas_call(
        body,
        grid=(bt, nblk),
        in_specs=[
            pl.BlockSpec((1, _P, d), lambda b, j: (b, j, 0)),
            pl.BlockSpec((_P, d), lambda b, j: (j, 0)),
            pl.BlockSpec((d, d), lambda b, j: (0, 0)),
            pl.BlockSpec((1, d), lambda b, j: (0, 0)),
        ],
        out_specs=pl.BlockSpec((1, _P, d), lambda b, j: (b, j, 0)),
        out_shape=jax.ShapeDtypeStruct((bt, nl, d), jnp.float32),
    )(sums, cnts, w, bias)


def kernel(x, current_level, target_level, parent_map, W_down, b_down):
    B, C, T, N, D = x.shape
    NL = N // 4
    BT = B * C * T

    xf = x.reshape(BT * N, D)
    pm32 = parent_map.astype(jnp.int32)
    pm_pad = jnp.concatenate([pm32, jnp.zeros((_CHUNK,), jnp.int32)])
    edges = jnp.arange(0, NL + 1, _P, dtype=jnp.int32)
    bounds = jnp.searchsorted(pm32, edges).astype(jnp.int32)
    bounds = jnp.repeat(bounds, 16)

    sums, cnts = _sc_segment_sums(xf, pm_pad, bounds, BT, N, NL, D)
    out = _tc_mean_conv(sums.reshape(BT, NL, D), cnts, W_down,
                        b_down.reshape(1, D), BT, NL, D)
    return out.reshape(B, C, T, NL, D)


# speedup vs baseline: 53.2046x; 1.9120x over previous
import jax
import jax.numpy as jnp
from jax import lax
from jax.experimental import pallas as pl
from jax.experimental.pallas import tpu as pltpu
from jax.experimental.pallas import tpu_sc as plsc

_NC = 2
_NS = 16
_NW = _NC * _NS

_P = 256
_CHUNK = 256
_NSUB = _CHUNK // 128
_S = 64
_TCP = 2048


def _vext(ref, i):
    return ref[pl.ds(i * 16, 16)][0]


def _sc_segment_sums(xf, pm_pad, bounds, bt, n, nl, d):
    nblk = nl // _P
    items = bt * nblk
    per_w = items // _NW

    def body(xf_hbm, pm_hbm, bounds_hbm, sums_hbm, cnts_hbm,
             acc_sh, cnt_sh, chunk, zbuf, pmbuf, idx2, onesb, boundsv,
             sem_gx, sem_gp, sem_s, sem_sc, sem_z, sem_zc, sem_d, sem_dc):
        cid = lax.axis_index("c")
        sid = lax.axis_index("s")
        wid = sid * _NC + cid
        pltpu.sync_copy(bounds_hbm, boundsv)

        ones16 = jnp.full((16,), 1.0, jnp.float32)
        zeros16 = jnp.zeros((16,), jnp.float32)

        @pl.loop(0, 128)
        def _fill_ones(r):
            for g in range(d // 16):
                onesb[r, pl.ds(g * 16, 16)] = ones16

        @pl.loop(0, (_P + 16) // 8)
        def _fill_zeros(r):
            for g in range(d // 16):
                zbuf[r, pl.ds(g * 16, 16)] = zeros16

        bank = _P + 16
        base = sid * bank
        eighth = bank // 8

        prev = {"acc": False, "cnt": False, "do_cnt": None}

        for k in range(per_w):
            t = wid * per_w + k
            b = t // nblk
            pb = t - b * nblk
            p0 = pb * _P
            r0 = jnp.maximum(_vext(boundsv, pb) - _S, 0)
            r1 = _vext(boundsv, pb + 1)
            do_cnt = ((b == 0) & (pb < nblk // 2)) | ((b == 1) & (pb >= nblk // 2))

            if prev["acc"]:
                pltpu.make_async_copy(
                    acc_sh.at[pl.ds(base, _P)],
                    sums_hbm.at[pl.ds(0, _P)], sem_d).wait()
            if prev["cnt"]:
                @pl.when(prev["do_cnt"])
                def _wait_cnt_drain():
                    pltpu.make_async_copy(
                        cnt_sh.at[pl.ds(base, _P)],
                        cnts_hbm.at[pl.ds(0, _P)], sem_dc).wait()

            for q in range(8):
                pltpu.async_copy(zbuf, acc_sh.at[pl.ds(base + q * eighth, eighth)],
                                 sem_z)

            @pl.when(do_cnt)
            def _zero_cnt():
                for q in range(8):
                    pltpu.async_copy(zbuf,
                                     cnt_sh.at[pl.ds(base + q * eighth, eighth)],
                                     sem_zc)

            a0 = (r0 // 8) * 8
            nch = (r1 - a0 + _CHUNK - 1) // _CHUNK
            rowbase = b * n

            @pl.loop(0, nch)
            def _chunk(ch):
                st = a0 + ch * _CHUNK
                gst = jnp.minimum(rowbase + st, bt * n - _CHUNK)
                st_c = gst - rowbase
                dp = pltpu.async_copy(pm_hbm.at[pl.ds(st_c, _CHUNK)], pmbuf,
                                      sem_gp)
                dx = pltpu.async_copy(xf_hbm.at[pl.ds(gst, _CHUNK)], chunk,
                                      sem_gx)
                dp.wait()
                vlo = rowbase + st
                vhi = rowbase + n
                for j16 in range(_CHUNK // 16):
                    pv = pmbuf[pl.ds(j16 * 16, 16)]
                    rows = gst + j16 * 16 + lax.iota(jnp.int32, 16)
                    valid = ((rows >= vlo) & (rows < vhi)
                             & (pv >= p0) & (pv < p0 + _P))
                    li = base + jnp.where(valid, pv - p0, _P)
                    idx2[j16 // 8, pl.ds((j16 % 8) * 16, 16)] = li
                dx.wait()

                @pl.when(ch == 0)
                def _wait_zero():
                    for q in range(8):
                        pltpu.make_async_copy(
                            zbuf, acc_sh.at[pl.ds(base, eighth)], sem_z).wait()

                @pl.when((ch == 0) & do_cnt)
                def _wait_zero_cnt():
                    for q in range(8):
                        pltpu.make_async_copy(
                            zbuf, cnt_sh.at[pl.ds(base, eighth)], sem_zc).wait()

                for j in range(_NSUB):
                    ds = pltpu.async_copy(chunk.at[pl.ds(j * 128, 128)],
                                          acc_sh.at[idx2.at[j]], sem_s,
                                          add=True)

                    @pl.when(do_cnt)
                    def _scatter_cnt():
                        pltpu.async_copy(onesb, cnt_sh.at[idx2.at[j]],
                                         sem_sc, add=True)

                    ds.wait()

                    @pl.when(do_cnt)
                    def _wait_cnt():
                        pltpu.make_async_copy(
                            onesb, cnt_sh.at[idx2.at[j]], sem_sc).wait()

            @pl.when(nch == 0)
            def _wait_zero_empty():
                for q in range(8):
                    pltpu.make_async_copy(
                        zbuf, acc_sh.at[pl.ds(base, eighth)], sem_z).wait()

            @pl.when((nch == 0) & do_cnt)
            def _wait_zero_cnt_empty():
                for q in range(8):
                    pltpu.make_async_copy(
                        zbuf, cnt_sh.at[pl.ds(base, eighth)], sem_zc).wait()

            q0 = b * nl + p0
            pltpu.async_copy(acc_sh.at[pl.ds(base, _P)],
                             sums_hbm.at[pl.ds(q0, _P)], sem_d)
            prev["acc"] = True

            @pl.when(do_cnt)
            def _store_counts():
                pltpu.async_copy(cnt_sh.at[pl.ds(base, _P)],
                                 cnts_hbm.at[pl.ds(p0, _P)], sem_dc)
            prev["cnt"] = True
            prev["do_cnt"] = do_cnt

        pltpu.make_async_copy(acc_sh.at[pl.ds(base, _P)],
                              sums_hbm.at[pl.ds(0, _P)], sem_d).wait()

        @pl.when(prev["do_cnt"])
        def _wait_final_cnt():
            pltpu.make_async_copy(cnt_sh.at[pl.ds(base, _P)],
                                  cnts_hbm.at[pl.ds(0, _P)], sem_dc).wait()

    mesh = plsc.VectorSubcoreMesh(core_axis_name="c", subcore_axis_name="s")
    return pl.kernel(
        body,
        out_type=(
            jax.ShapeDtypeStruct((bt * nl, d), jnp.float32),
            jax.ShapeDtypeStruct((nl, d), jnp.float32),
        ),
        mesh=mesh,
        scratch_types=[
            pltpu.VMEM_SHARED((_NS * (_P + 16), d), jnp.float32),
            pltpu.VMEM_SHARED((_NS * (_P + 16), d), jnp.float32),
            pltpu.VMEM((_CHUNK, d), jnp.float32),
            pltpu.VMEM(((_P + 16) // 8, d), jnp.float32),
            pltpu.VMEM((_CHUNK,), jnp.int32),
            pltpu.VMEM((_NSUB, 128), jnp.int32),
            pltpu.VMEM((128, d), jnp.float32),
            pltpu.VMEM(((nl // _P + 1) * 16,), jnp.int32),
            pltpu.SemaphoreType.DMA,
            pltpu.SemaphoreType.DMA,
            pltpu.SemaphoreType.DMA,
            pltpu.SemaphoreType.DMA,
            pltpu.SemaphoreType.DMA,
            pltpu.SemaphoreType.DMA,
            pltpu.SemaphoreType.DMA,
            pltpu.SemaphoreType.DMA,
        ],
    )(xf, pm_pad, bounds)


def _tc_mean_conv(sums, cnts, w, bias, bt, nl, d):
    nblk = nl // _TCP

    def body(s_ref, c_ref, w_ref, b_ref, o_ref):
        s = s_ref[0]
        cnt = jnp.maximum(c_ref[:, 0:1], 1e-6)
        r = s / cnt
        y = lax.dot_general(r, w_ref[...], (((1,), (1,)), ((), ())),
                            preferred_element_type=jnp.float32)
        o_ref[0] = y + b_ref[...]

    return pl.pallas_call(
        body,
        grid=(bt, nblk),
        in_specs=[
            pl.BlockSpec((1, _TCP, d), lambda b, j: (b, j, 0)),
            pl.BlockSpec((_TCP, d), lambda b, j: (j, 0)),
            pl.BlockSpec((d, d), lambda b, j: (0, 0)),
            pl.BlockSpec((1, d), lambda b, j: (0, 0)),
        ],
        out_specs=pl.BlockSpec((1, _TCP, d), lambda b, j: (b, j, 0)),
        out_shape=jax.ShapeDtypeStruct((bt, nl, d), jnp.float32),
    )(sums, cnts, w, bias)


def kernel(x, current_level, target_level, parent_map, W_down, b_down):
    B, C, T, N, D = x.shape
    NL = N // 4
    BT = B * C * T

    xf = x.reshape(BT * N, D)
    pm32 = parent_map.astype(jnp.int32)
    pm_pad = jnp.concatenate([pm32, jnp.zeros((_CHUNK,), jnp.int32)])
    edges = jnp.arange(0, NL + 1, _P, dtype=jnp.int32)
    pm_s = pm32[::_S]
    k = jnp.sum((pm_s[None, :] < edges[:, None]).astype(jnp.int32), axis=1)
    bounds = (k * _S).astype(jnp.int32)
    bounds = jnp.repeat(bounds, 16)

    sums, cnts = _sc_segment_sums(xf, pm_pad, bounds, BT, N, NL, D)
    out = _tc_mean_conv(sums.reshape(BT, NL, D), cnts, W_down,
                        b_down.reshape(1, D), BT, NL, D)
    return out.reshape(B, C, T, NL, D)
